# baseline (speedup 1.0000x reference)
import functools

import jax
import jax.numpy as jnp
from jax import lax
from jax.experimental import pallas as pl
from jax.experimental.pallas import tpu as pltpu
from jax.experimental.pallas import tpu_sc as plsc

B = 16384
L = 200
NC = 2
NS = 16
NW = NC * NS
ROWS_PER_W = B // NW
CHUNK = 16
NCHUNK = ROWS_PER_W // CHUNK


def _sc_body(x_hbm, w_hbm, out_hbm, xc, wv, outv):
    wid = lax.axis_index("s") * NC + lax.axis_index("c")
    base = wid * ROWS_PER_W
    pltpu.sync_copy(w_hbm, wv)
    row_base = lax.iota(jnp.int32, 16) * L

    def chunk_body(c, _):
        pltpu.sync_copy(
            x_hbm.at[pl.ds((base + c * CHUNK) * L, CHUNK * L)], xc)
        acc = jnp.zeros((16,), jnp.float32)
        for l in range(L):
            v = plsc.load_gather(xc, [row_base + l])
            acc = acc + plsc.load_gather(wv, [v])
        outv[pl.ds(c * CHUNK, CHUNK)] = acc
        return 0

    lax.fori_loop(0, NCHUNK, chunk_body, 0)
    pltpu.sync_copy(outv, out_hbm.at[pl.ds(base, ROWS_PER_W)])


@jax.jit
def _sc_call(x, w_pad):
    mesh = plsc.VectorSubcoreMesh(core_axis_name="c", subcore_axis_name="s")
    f = pl.kernel(
        _sc_body,
        out_type=jax.ShapeDtypeStruct((B,), jnp.float32),
        mesh=mesh,
        scratch_types=[
            pltpu.VMEM((CHUNK * L,), jnp.int32),
            pltpu.VMEM((16,), jnp.float32),
            pltpu.VMEM((ROWS_PER_W,), jnp.float32),
        ],
        compiler_params=pltpu.CompilerParams(
            use_tc_tiling_on_sc=False, needs_layout_passes=False),
    )
    return f(x, w_pad)


def kernel(x, emb_weight):
    w_pad = jnp.zeros((16,), jnp.float32).at[:5].set(emb_weight[:, 0])
    out = _sc_call(x.reshape(B * L), w_pad)
    return out.reshape(B, 1)


# speedup vs baseline: 1.1932x; 1.1932x over previous
import jax
import jax.numpy as jnp
from jax import lax
from jax.experimental import pallas as pl
from jax.experimental.pallas import tpu as pltpu
from jax.experimental.pallas import tpu_sc as plsc

B = 16384
L = 200
NC = 2
NS = 16
NW = NC * NS
ROWS_PER_W = B // NW
CHUNK = 16
NCHUNK = ROWS_PER_W // CHUNK
CHUNK_WORDS = CHUNK * L
TSTRIDE = 17
NFULL = L // 16
TAIL = L - NFULL * 16


def _sc_body(x_hbm, w_hbm, out_hbm, xb0, xb1, wv, outv, sem0, sem1):
    wid = lax.axis_index("s") * NC + lax.axis_index("c")
    base = wid * ROWS_PER_W
    pltpu.sync_copy(w_hbm, wv)

    lane = lax.iota(jnp.int32, 16)
    tbase = lane * TSTRIDE
    tail_mask = lane < TAIL

    def issue(c, buf, sem):
        pltpu.async_copy(
            x_hbm.at[pl.ds((base + c * CHUNK) * L, CHUNK_WORDS)],
            buf.at[pl.ds(0, CHUNK_WORDS)], sem)

    def drain(buf, sem):
        pltpu.make_async_copy(
            x_hbm.at[pl.ds(0, CHUNK_WORDS)],
            buf.at[pl.ds(0, CHUNK_WORDS)], sem).wait()

    def compute(c, buf):
        svec = jnp.zeros((16,), jnp.float32)
        for r in range(CHUNK):
            row_off = r * L
            acc = jnp.zeros((16,), jnp.float32)
            for j in range(NFULL):
                v = buf[pl.ds(row_off + j * 16, 16)]
                acc = acc + plsc.load_gather(wv, [v + tbase])
            v = buf[pl.ds(row_off + NFULL * 16, 16)]
            v = jnp.where(tail_mask, v, 16)
            acc = acc + plsc.load_gather(wv, [v + tbase])
            svec = jnp.where(lane == r, jnp.sum(acc), svec)
        outv[pl.ds(c * CHUNK, CHUNK)] = svec

    issue(0, xb0, sem0)

    def pair(i, _):
        c0 = 2 * i
        issue(c0 + 1, xb1, sem1)
        drain(xb0, sem0)
        compute(c0, xb0)

        @pl.when(c0 + 2 < NCHUNK)
        def _():
            issue(c0 + 2, xb0, sem0)

        drain(xb1, sem1)
        compute(c0 + 1, xb1)
        return 0

    lax.fori_loop(0, NCHUNK // 2, pair, 0)
    pltpu.sync_copy(outv, out_hbm.at[pl.ds(base, ROWS_PER_W)])


@jax.jit
def _sc_call(x_flat, w_rep):
    mesh = plsc.VectorSubcoreMesh(core_axis_name="c", subcore_axis_name="s")
    f = pl.kernel(
        _sc_body,
        out_type=jax.ShapeDtypeStruct((B,), jnp.float32),
        mesh=mesh,
        scratch_types=[
            pltpu.VMEM((CHUNK_WORDS + 16,), jnp.int32),
            pltpu.VMEM((CHUNK_WORDS + 16,), jnp.int32),
            pltpu.VMEM((16 * TSTRIDE,), jnp.float32),
            pltpu.VMEM((ROWS_PER_W,), jnp.float32),
            pltpu.SemaphoreType.DMA,
            pltpu.SemaphoreType.DMA,
        ],
        compiler_params=pltpu.CompilerParams(
            use_tc_tiling_on_sc=False, needs_layout_passes=False),
    )
    return f(x_flat, w_rep)


def kernel(x, emb_weight):
    w_pad = jnp.concatenate(
        [emb_weight[:, 0], jnp.zeros((TSTRIDE - 5,), jnp.float32)])
    w_rep = jnp.tile(w_pad, 16)
    out = _sc_call(x.reshape(B * L), w_rep)
    return out.reshape(B, 1)


# speedup vs baseline: 1.6862x; 1.4132x over previous
import jax
import jax.numpy as jnp
from jax import lax
from jax.experimental import pallas as pl
from jax.experimental.pallas import tpu as pltpu
from jax.experimental.pallas import tpu_sc as plsc

B = 16384
L = 200
NC = 2
NS = 16
NW = NC * NS
ROWS_PER_W = B // NW
CHUNK = 16
NCHUNK = ROWS_PER_W // CHUNK
TSTRIDE = 17
NFULL = L // 16
TAIL = L - NFULL * 16


def _sc_body(x_hbm, w_hbm, out_hbm, xb0, xb1, wv, outv, sem0, sem1):
    wid = lax.axis_index("s") * NC + lax.axis_index("c")
    base = wid * ROWS_PER_W
    pltpu.sync_copy(w_hbm, wv)

    lane = lax.iota(jnp.int32, 16)
    tbase = lane * TSTRIDE
    dup_mask = lane >= TAIL

    def issue(c, buf, sem):
        pltpu.async_copy(x_hbm.at[pl.ds(base + c * CHUNK, CHUNK)], buf, sem)

    def drain(buf, sem):
        pltpu.make_async_copy(
            x_hbm.at[pl.ds(base, CHUNK)], buf, sem).wait()

    def compute(c, buf):
        svec = jnp.zeros((16,), jnp.float32)
        for r in range(CHUNK):
            acc = jnp.zeros((16,), jnp.float32)
            for j in range(NFULL):
                v = buf[r, pl.ds(j * 16, 16)]
                acc = acc + plsc.load_gather(wv, [v + tbase])
            v = buf[r, pl.ds(L - 16, 16)]
            v = jnp.where(dup_mask, v, 16)
            acc = acc + plsc.load_gather(wv, [v + tbase])
            svec = jnp.where(lane == r, jnp.sum(acc), svec)
        outv[pl.ds(c * CHUNK, CHUNK)] = svec

    issue(0, xb0, sem0)

    def pair(i, _):
        c0 = 2 * i
        issue(c0 + 1, xb1, sem1)
        drain(xb0, sem0)
        compute(c0, xb0)

        @pl.when(c0 + 2 < NCHUNK)
        def _():
            issue(c0 + 2, xb0, sem0)

        drain(xb1, sem1)
        compute(c0 + 1, xb1)
        return 0

    lax.fori_loop(0, NCHUNK // 2, pair, 0)
    pltpu.sync_copy(outv, out_hbm.at[pl.ds(base, ROWS_PER_W)])


@jax.jit
def _sc_call(x, w_rep):
    mesh = plsc.VectorSubcoreMesh(core_axis_name="c", subcore_axis_name="s")
    f = pl.kernel(
        _sc_body,
        out_type=jax.ShapeDtypeStruct((B,), jnp.float32),
        mesh=mesh,
        scratch_types=[
            pltpu.VMEM((CHUNK, L), jnp.int32),
            pltpu.VMEM((CHUNK, L), jnp.int32),
            pltpu.VMEM((16 * TSTRIDE,), jnp.float32),
            pltpu.VMEM((ROWS_PER_W,), jnp.float32),
            pltpu.SemaphoreType.DMA,
            pltpu.SemaphoreType.DMA,
        ],
        compiler_params=pltpu.CompilerParams(
            use_tc_tiling_on_sc=True, needs_layout_passes=False),
    )
    return f(x, w_rep)


def kernel(x, emb_weight):
    w_pad = jnp.concatenate(
        [emb_weight[:, 0], jnp.zeros((TSTRIDE - 5,), jnp.float32)])
    w_rep = jnp.tile(w_pad, 16)
    out = _sc_call(x, w_rep)
    return out.reshape(B, 1)


# speedup vs baseline: 2.5032x; 1.4845x over previous
import jax
import jax.numpy as jnp
from jax import lax
from jax.experimental import pallas as pl
from jax.experimental.pallas import tpu as pltpu
from jax.experimental.pallas import tpu_sc as plsc

B = 16384
L = 200
NC = 2
NS = 16
NW = NC * NS
COLS_PER_W = B // NW
NGROUP = COLS_PER_W // 16
LCHUNK = 8
NCHUNK = L // LCHUNK
PIPE = 4
TSTRIDE = 17


def _sc_body(x_hbm, w_hbm, out_hbm, bb, wv, accv, sem):
    wid = lax.axis_index("s") * NC + lax.axis_index("c")
    base = wid * COLS_PER_W
    pltpu.sync_copy(w_hbm, wv)

    tbase = lax.iota(jnp.int32, 16) * TSTRIDE

    def issue(i):
        pltpu.async_copy(
            x_hbm.at[pl.ds(i * LCHUNK, LCHUNK), pl.ds(base, COLS_PER_W)],
            bb.at[pl.ds(i * LCHUNK, LCHUNK), :], sem)

    def zero_group(g, _):
        accv[pl.ds(g * 16, 16)] = jnp.zeros((16,), jnp.float32)
        return 0

    lax.fori_loop(0, NGROUP, zero_group, 0)

    for i in range(PIPE):
        issue(i)

    def chunk_body(i, _):
        pltpu.make_async_copy(
            x_hbm.at[pl.ds(0, LCHUNK), pl.ds(base, COLS_PER_W)],
            bb.at[pl.ds(0, LCHUNK), :], sem).wait()

        def group_body(g, _):
            gl = g * 16
            part = jnp.zeros((16,), jnp.float32)
            for l in range(LCHUNK):
                v = bb[i * LCHUNK + l, pl.ds(gl, 16)]
                part = part + plsc.load_gather(wv, [v + tbase])
            plsc.addupdate(accv.at[pl.ds(gl, 16)], part)
            return 0

        lax.fori_loop(0, NGROUP, group_body, 0)

        @pl.when(i + PIPE < NCHUNK)
        def _():
            issue(i + PIPE)

        return 0

    lax.fori_loop(0, NCHUNK, chunk_body, 0)
    pltpu.sync_copy(accv, out_hbm.at[pl.ds(base, COLS_PER_W)])


@jax.jit
def _sc_call(x_t, w_rep):
    mesh = plsc.VectorSubcoreMesh(core_axis_name="c", subcore_axis_name="s")
    f = pl.kernel(
        _sc_body,
        out_type=jax.ShapeDtypeStruct((B,), jnp.float32),
        mesh=mesh,
        scratch_types=[
            pltpu.VMEM((L, COLS_PER_W), jnp.int32),
            pltpu.VMEM((16 * TSTRIDE,), jnp.float32),
            pltpu.VMEM((COLS_PER_W,), jnp.float32),
            pltpu.SemaphoreType.DMA,
        ],
        compiler_params=pltpu.CompilerParams(
            use_tc_tiling_on_sc=True, needs_layout_passes=False),
    )
    return f(x_t, w_rep)


def kernel(x, emb_weight):
    w_pad = jnp.concatenate(
        [emb_weight[:, 0], jnp.zeros((TSTRIDE - 5,), jnp.float32)])
    w_rep = jnp.tile(w_pad, 16)
    out = _sc_call(x.T, w_rep)
    return out.reshape(B, 1)
